# baseline (speedup 1.0000x reference)
import functools
import math

import jax
import jax.numpy as jnp
import numpy as np
from jax import lax
from jax.experimental import pallas as pl
from jax.experimental.pallas import tpu as pltpu
from jax.experimental.pallas import tpu_sc as plsc

MAX_LEN = 512
DIM = 1024
EPS = 1e-5

L = 16
NW = 32
CHUNK = 32
NCHUNK = MAX_LEN // CHUNK


def _pe_table() -> jnp.ndarray:
    position = np.arange(0, MAX_LEN, dtype=np.float32)[:, None]
    div_term = np.exp(
        np.arange(0, DIM, 2, dtype=np.float32) * -(math.log(10000.0) / DIM))
    pe = np.zeros((MAX_LEN, DIM), dtype=np.float32)
    pe[:, 0::2] = np.sin(position * div_term)
    pe[:, 1::2] = np.cos(position * div_term)
    return jnp.asarray(pe)


def _lane_allsum(v, perms):
    for p in perms:
        v = v + v.at[p].get(mode="promise_in_bounds")
    return v


def _rsqrt(x):
    i = lax.bitcast_convert_type(x, jnp.int32)
    i = jnp.int32(0x5F3759DF) - lax.shift_right_arithmetic(i, 1)
    y = lax.bitcast_convert_type(i, jnp.float32)
    for _ in range(3):
        y = y * (jnp.float32(1.5) - jnp.float32(0.5) * x * y * y)
    return y


def _sc_body(pe_hbm, pidx_hbm, sidx_hbm, gamma_hbm, beta_hbm, out_hbm,
             idxp_v, idxs_v, bufp_v, bufs_v, bufj_v, gamma_v, beta_v,
             semp, sems):
    wid = lax.axis_index("s") * 2 + lax.axis_index("c")

    pltpu.sync_copy(pidx_hbm.at[wid], idxp_v)
    pltpu.sync_copy(sidx_hbm.at[wid], idxs_v)
    pltpu.sync_copy(gamma_hbm, gamma_v)
    pltpu.sync_copy(beta_hbm, beta_v)

    lanes = lax.iota(jnp.int32, L)
    perms = [lax.bitwise_xor(lanes, jnp.int32(sh)) for sh in (1, 2, 4, 8)]

    def chunk_step(c, _):
        gp = pltpu.async_copy(pe_hbm.at[idxp_v.at[c]], bufp_v, semp)
        gs = pltpu.async_copy(pe_hbm.at[idxs_v.at[c]], bufs_v, sems)
        pltpu.sync_copy(pe_hbm.at[pl.ds(c * CHUNK, CHUNK)], bufj_v)
        gp.wait()
        gs.wait()

        def row_step(r, _):
            def acc_step(k, carry):
                ssum, ssq = carry
                v = (bufp_v[r, pl.ds(k * L, L)]
                     + bufs_v[r, pl.ds(k * L, L)]
                     + bufj_v[r, pl.ds(k * L, L)])
                bufp_v[r, pl.ds(k * L, L)] = v
                return ssum + v, ssq + v * v

            zeros = jnp.zeros((L,), jnp.float32)
            ssum, ssq = lax.fori_loop(0, DIM // L, acc_step, (zeros, zeros))
            mean = _lane_allsum(ssum, perms) * jnp.float32(1.0 / DIM)
            var = _lane_allsum(ssq, perms) * jnp.float32(1.0 / DIM) - mean * mean
            rstd = _rsqrt(var + jnp.float32(EPS))

            def norm_step(k, _):
                v = bufp_v[r, pl.ds(k * L, L)]
                g = gamma_v[pl.ds(k * L, L)]
                b = beta_v[pl.ds(k * L, L)]
                bufp_v[r, pl.ds(k * L, L)] = (v - mean) * rstd * g + b
                return 0

            lax.fori_loop(0, DIM // L, norm_step, 0)
            return 0

        lax.fori_loop(0, CHUNK, row_step, 0)
        pltpu.sync_copy(bufp_v, out_hbm.at[pl.ds(wid * MAX_LEN + c * CHUNK,
                                                 CHUNK)])
        return 0

    lax.fori_loop(0, NCHUNK, chunk_step, 0)


@jax.jit
def _run(pidx, sidx, ln_gamma, ln_beta):
    pe = _pe_table()
    mesh = plsc.VectorSubcoreMesh(core_axis_name="c", subcore_axis_name="s")
    f = pl.kernel(
        _sc_body,
        out_type=jax.ShapeDtypeStruct((NW * MAX_LEN, DIM), jnp.float32),
        mesh=mesh,
        scratch_types=[
            pltpu.VMEM((NCHUNK, CHUNK), jnp.int32),
            pltpu.VMEM((NCHUNK, CHUNK), jnp.int32),
            pltpu.VMEM((CHUNK, DIM), jnp.float32),
            pltpu.VMEM((CHUNK, DIM), jnp.float32),
            pltpu.VMEM((CHUNK, DIM), jnp.float32),
            pltpu.VMEM((DIM,), jnp.float32),
            pltpu.VMEM((DIM,), jnp.float32),
            pltpu.SemaphoreType.DMA,
            pltpu.SemaphoreType.DMA,
        ],
    )
    return f(pe, pidx, sidx, ln_gamma, ln_beta)


def kernel(top_vecs, sent_struct_vec, ln_gamma, ln_beta):
    B, n, _ = top_vecs.shape
    idx = sent_struct_vec.astype(jnp.int32)
    pidx = idx[:, :, 0].reshape(B, NCHUNK, CHUNK)
    sidx = idx[:, :, 1].reshape(B, NCHUNK, CHUNK)
    out = _run(pidx, sidx, ln_gamma.astype(jnp.float32),
               ln_beta.astype(jnp.float32))
    return out.reshape(B, n, DIM)


# speedup vs baseline: 1.4037x; 1.4037x over previous
import functools
import math

import jax
import jax.numpy as jnp
import numpy as np
from jax import lax
from jax.experimental import pallas as pl
from jax.experimental.pallas import tpu as pltpu
from jax.experimental.pallas import tpu_sc as plsc

MAX_LEN = 512
DIM = 1024
EPS = 1e-5

L = 16
NW = 32
CHUNK = 32
NCHUNK = MAX_LEN // CHUNK


def _pe_table() -> jnp.ndarray:
    position = np.arange(0, MAX_LEN, dtype=np.float32)[:, None]
    div_term = np.exp(
        np.arange(0, DIM, 2, dtype=np.float32) * -(math.log(10000.0) / DIM))
    pe = np.zeros((MAX_LEN, DIM), dtype=np.float32)
    pe[:, 0::2] = np.sin(position * div_term)
    pe[:, 1::2] = np.cos(position * div_term)
    return jnp.asarray(pe)


def _lane_allsum(v, perms):
    for p in perms:
        v = v + v.at[p].get(mode="promise_in_bounds")
    return v


def _rsqrt(x):
    i = lax.bitcast_convert_type(x, jnp.int32)
    i = jnp.int32(0x5F3759DF) - lax.shift_right_arithmetic(i, 1)
    y = lax.bitcast_convert_type(i, jnp.float32)
    for _ in range(3):
        y = y * (jnp.float32(1.5) - jnp.float32(0.5) * x * y * y)
    return y


def _sc_body(pe_hbm, pidx_hbm, sidx_hbm, gamma_hbm, beta_hbm, out_hbm,
             idxp_v, idxs_v, bufp_v, bufs_v, bufj_v, gamma_v, beta_v,
             semp, sems):
    wid = lax.axis_index("s") * 2 + lax.axis_index("c")

    pltpu.sync_copy(pidx_hbm.at[wid], idxp_v)
    pltpu.sync_copy(sidx_hbm.at[wid], idxs_v)
    pltpu.sync_copy(gamma_hbm, gamma_v)
    pltpu.sync_copy(beta_hbm, beta_v)

    lanes = lax.iota(jnp.int32, L)
    perms = [lax.bitwise_xor(lanes, jnp.int32(sh)) for sh in (1, 2, 4, 8)]

    def chunk_step(c, _):
        gp = pltpu.async_copy(pe_hbm.at[idxp_v.at[c]], bufp_v, semp)
        gs = pltpu.async_copy(pe_hbm.at[idxs_v.at[c]], bufs_v, sems)
        pltpu.sync_copy(pe_hbm.at[pl.ds(c * CHUNK, CHUNK)], bufj_v)
        gp.wait()
        gs.wait()

        def row_step(r, _):
            ssum = jnp.zeros((L,), jnp.float32)
            ssq = jnp.zeros((L,), jnp.float32)
            for k in range(DIM // L):
                v = (bufp_v[r, pl.ds(k * L, L)]
                     + bufs_v[r, pl.ds(k * L, L)]
                     + bufj_v[r, pl.ds(k * L, L)])
                bufp_v[r, pl.ds(k * L, L)] = v
                ssum = ssum + v
                ssq = ssq + v * v
            mean = _lane_allsum(ssum, perms) * jnp.float32(1.0 / DIM)
            var = _lane_allsum(ssq, perms) * jnp.float32(1.0 / DIM) - mean * mean
            rstd = _rsqrt(var + jnp.float32(EPS))

            for k in range(DIM // L):
                v = bufp_v[r, pl.ds(k * L, L)]
                g = gamma_v[pl.ds(k * L, L)]
                b = beta_v[pl.ds(k * L, L)]
                bufp_v[r, pl.ds(k * L, L)] = (v - mean) * rstd * g + b
            return 0

        lax.fori_loop(0, CHUNK, row_step, 0)
        pltpu.sync_copy(bufp_v, out_hbm.at[pl.ds(wid * MAX_LEN + c * CHUNK,
                                                 CHUNK)])
        return 0

    lax.fori_loop(0, NCHUNK, chunk_step, 0)


@jax.jit
def _run(pidx, sidx, ln_gamma, ln_beta):
    pe = _pe_table()
    mesh = plsc.VectorSubcoreMesh(core_axis_name="c", subcore_axis_name="s")
    f = pl.kernel(
        _sc_body,
        out_type=jax.ShapeDtypeStruct((NW * MAX_LEN, DIM), jnp.float32),
        mesh=mesh,
        scratch_types=[
            pltpu.VMEM((NCHUNK, CHUNK), jnp.int32),
            pltpu.VMEM((NCHUNK, CHUNK), jnp.int32),
            pltpu.VMEM((CHUNK, DIM), jnp.float32),
            pltpu.VMEM((CHUNK, DIM), jnp.float32),
            pltpu.VMEM((CHUNK, DIM), jnp.float32),
            pltpu.VMEM((DIM,), jnp.float32),
            pltpu.VMEM((DIM,), jnp.float32),
            pltpu.SemaphoreType.DMA,
            pltpu.SemaphoreType.DMA,
        ],
    )
    return f(pe, pidx, sidx, ln_gamma, ln_beta)


def kernel(top_vecs, sent_struct_vec, ln_gamma, ln_beta):
    B, n, _ = top_vecs.shape
    idx = sent_struct_vec.astype(jnp.int32)
    pidx = idx[:, :, 0].reshape(B, NCHUNK, CHUNK)
    sidx = idx[:, :, 1].reshape(B, NCHUNK, CHUNK)
    out = _run(pidx, sidx, ln_gamma.astype(jnp.float32),
               ln_beta.astype(jnp.float32))
    return out.reshape(B, n, DIM)


# speedup vs baseline: 1.5145x; 1.0789x over previous
import functools
import math

import jax
import jax.numpy as jnp
import numpy as np
from jax import lax
from jax.experimental import pallas as pl
from jax.experimental.pallas import tpu as pltpu
from jax.experimental.pallas import tpu_sc as plsc

MAX_LEN = 512
DIM = 1024
EPS = 1e-5

L = 16
NW = 32
CHUNK = 16
NCHUNK = MAX_LEN // CHUNK


def _pe_table() -> jnp.ndarray:
    position = np.arange(0, MAX_LEN, dtype=np.float32)[:, None]
    div_term = np.exp(
        np.arange(0, DIM, 2, dtype=np.float32) * -(math.log(10000.0) / DIM))
    pe = np.zeros((MAX_LEN, DIM), dtype=np.float32)
    pe[:, 0::2] = np.sin(position * div_term)
    pe[:, 1::2] = np.cos(position * div_term)
    return jnp.asarray(pe)


def _lane_allsum(v, perms):
    for p in perms:
        v = v + v.at[p].get(mode="promise_in_bounds")
    return v


def _rsqrt(x):
    i = lax.bitcast_convert_type(x, jnp.int32)
    i = jnp.int32(0x5F3759DF) - lax.shift_right_arithmetic(i, 1)
    y = lax.bitcast_convert_type(i, jnp.float32)
    for _ in range(3):
        y = y * (jnp.float32(1.5) - jnp.float32(0.5) * x * y * y)
    return y


def _sc_body(pe_hbm, pidx_hbm, sidx_hbm, gamma_hbm, beta_hbm, out_hbm,
             idxp_v, idxs_v, bufp_v, bufs_v, bufj_v, gamma_v, beta_v,
             semg, semo):
    wid = lax.axis_index("s") * 2 + lax.axis_index("c")

    pltpu.sync_copy(pidx_hbm.at[wid], idxp_v)
    pltpu.sync_copy(sidx_hbm.at[wid], idxs_v)
    pltpu.sync_copy(gamma_hbm, gamma_v)
    pltpu.sync_copy(beta_hbm, beta_v)

    lanes = lax.iota(jnp.int32, L)
    perms = [lax.bitwise_xor(lanes, jnp.int32(sh)) for sh in (1, 2, 4, 8)]

    def start_gathers(c, h):
        pltpu.make_async_copy(
            pe_hbm.at[idxp_v.at[c]], bufp_v.at[h], semg.at[h]).start()
        pltpu.make_async_copy(
            pe_hbm.at[idxs_v.at[c]], bufs_v.at[h], semg.at[h]).start()
        pltpu.make_async_copy(
            pe_hbm.at[pl.ds(c * CHUNK, CHUNK)], bufj_v.at[h], semg.at[h]
        ).start()

    def wait_gathers(c, h):
        pltpu.make_async_copy(
            pe_hbm.at[idxp_v.at[c]], bufp_v.at[h], semg.at[h]).wait()
        pltpu.make_async_copy(
            pe_hbm.at[idxs_v.at[c]], bufs_v.at[h], semg.at[h]).wait()
        pltpu.make_async_copy(
            pe_hbm.at[pl.ds(c * CHUNK, CHUNK)], bufj_v.at[h], semg.at[h]
        ).wait()

    def out_copy(c, h):
        return pltpu.make_async_copy(
            bufp_v.at[h],
            out_hbm.at[pl.ds(wid * MAX_LEN + c * CHUNK, CHUNK)],
            semo.at[h])

    def compute_chunk(h):
        def row_step(r, _):
            ssum = jnp.zeros((L,), jnp.float32)
            ssq = jnp.zeros((L,), jnp.float32)
            for k in range(DIM // L):
                v = (bufp_v[h, r, pl.ds(k * L, L)]
                     + bufs_v[h, r, pl.ds(k * L, L)]
                     + bufj_v[h, r, pl.ds(k * L, L)])
                bufp_v[h, r, pl.ds(k * L, L)] = v
                ssum = ssum + v
                ssq = ssq + v * v
            mean = _lane_allsum(ssum, perms) * jnp.float32(1.0 / DIM)
            var = (_lane_allsum(ssq, perms) * jnp.float32(1.0 / DIM)
                   - mean * mean)
            rstd = _rsqrt(var + jnp.float32(EPS))

            for k in range(DIM // L):
                v = bufp_v[h, r, pl.ds(k * L, L)]
                g = gamma_v[pl.ds(k * L, L)]
                b = beta_v[pl.ds(k * L, L)]
                bufp_v[h, r, pl.ds(k * L, L)] = (v - mean) * rstd * g + b
            return 0

        lax.fori_loop(0, CHUNK, row_step, 0)

    start_gathers(0, 0)

    def step(i, h):
        c = 2 * i + h

        @pl.when(c >= 1)
        def _():
            out_copy(c - 1, 1 - h).wait()

        @pl.when(c + 1 < NCHUNK)
        def _():
            start_gathers(c + 1, 1 - h)

        wait_gathers(c, h)
        compute_chunk(h)
        out_copy(c, h).start()

    def body(i, _):
        step(i, 0)
        step(i, 1)
        return 0

    lax.fori_loop(0, NCHUNK // 2, body, 0)
    out_copy(NCHUNK - 1, 1).wait()


@jax.jit
def _run(pidx, sidx, ln_gamma, ln_beta):
    pe = _pe_table()
    mesh = plsc.VectorSubcoreMesh(core_axis_name="c", subcore_axis_name="s")
    f = pl.kernel(
        _sc_body,
        out_type=jax.ShapeDtypeStruct((NW * MAX_LEN, DIM), jnp.float32),
        mesh=mesh,
        scratch_types=[
            pltpu.VMEM((NCHUNK, CHUNK), jnp.int32),
            pltpu.VMEM((NCHUNK, CHUNK), jnp.int32),
            pltpu.VMEM((2, CHUNK, DIM), jnp.float32),
            pltpu.VMEM((2, CHUNK, DIM), jnp.float32),
            pltpu.VMEM((2, CHUNK, DIM), jnp.float32),
            pltpu.VMEM((DIM,), jnp.float32),
            pltpu.VMEM((DIM,), jnp.float32),
            pltpu.SemaphoreType.DMA((2,)),
            pltpu.SemaphoreType.DMA((2,)),
        ],
    )
    return f(pe, pidx, sidx, ln_gamma, ln_beta)


def kernel(top_vecs, sent_struct_vec, ln_gamma, ln_beta):
    B, n, _ = top_vecs.shape
    idx = sent_struct_vec.astype(jnp.int32)
    pidx = idx[:, :, 0].reshape(B, NCHUNK, CHUNK)
    sidx = idx[:, :, 1].reshape(B, NCHUNK, CHUNK)
    out = _run(pidx, sidx, ln_gamma.astype(jnp.float32),
               ln_beta.astype(jnp.float32))
    return out.reshape(B, n, DIM)


# speedup vs baseline: 2.7669x; 1.8270x over previous
import functools
import math

import jax
import jax.numpy as jnp
import numpy as np
from jax import lax
from jax.experimental import pallas as pl
from jax.experimental.pallas import tpu as pltpu
from jax.experimental.pallas import tpu_sc as plsc

MAX_LEN = 512
DIM = 1024
EPS = 1e-5

L = 16
NW = 32
CHUNK = 16
NCHUNK = MAX_LEN // CHUNK
ROWBLK = 256


def _pe_table() -> jnp.ndarray:
    position = np.arange(0, MAX_LEN, dtype=np.float32)[:, None]
    div_term = np.exp(
        np.arange(0, DIM, 2, dtype=np.float32) * -(math.log(10000.0) / DIM))
    pe = np.zeros((MAX_LEN, DIM), dtype=np.float32)
    pe[:, 0::2] = np.sin(position * div_term)
    pe[:, 1::2] = np.cos(position * div_term)
    return jnp.asarray(pe)


def _sc_body(pe_hbm, pidx_hbm, sidx_hbm, out_hbm,
             idxp_v, idxs_v, bufp_v, bufs_v, semg, semo):
    wid = lax.axis_index("s") * 2 + lax.axis_index("c")

    pltpu.sync_copy(pidx_hbm.at[wid], idxp_v)
    pltpu.sync_copy(sidx_hbm.at[wid], idxs_v)

    def start_gathers(c, h):
        pltpu.make_async_copy(
            pe_hbm.at[idxp_v.at[c]], bufp_v.at[h], semg.at[h]).start()
        pltpu.make_async_copy(
            pe_hbm.at[idxs_v.at[c]], bufs_v.at[h], semg.at[h]).start()

    def wait_gathers(c, h):
        pltpu.make_async_copy(
            pe_hbm.at[idxp_v.at[c]], bufp_v.at[h], semg.at[h]).wait()
        pltpu.make_async_copy(
            pe_hbm.at[idxs_v.at[c]], bufs_v.at[h], semg.at[h]).wait()

    def out_copy(c, h):
        return pltpu.make_async_copy(
            bufp_v.at[h],
            out_hbm.at[pl.ds(wid * MAX_LEN + c * CHUNK, CHUNK)],
            semo.at[h])

    def compute_chunk(h):
        def row_step(r, _):
            for k in range(DIM // L):
                bufp_v[h, r, pl.ds(k * L, L)] = (
                    bufp_v[h, r, pl.ds(k * L, L)]
                    + bufs_v[h, r, pl.ds(k * L, L)])
            return 0

        lax.fori_loop(0, CHUNK, row_step, 0)

    start_gathers(0, 0)

    def step(i, h):
        c = 2 * i + h

        @pl.when(c >= 1)
        def _():
            out_copy(c - 1, 1 - h).wait()

        @pl.when(c + 1 < NCHUNK)
        def _():
            start_gathers(c + 1, 1 - h)

        wait_gathers(c, h)
        compute_chunk(h)
        out_copy(c, h).start()

    def body(i, _):
        step(i, 0)
        step(i, 1)
        return 0

    lax.fori_loop(0, NCHUNK // 2, body, 0)
    out_copy(NCHUNK - 1, 1).wait()


def _tc_ln_body(e2_ref, pe_ref, gamma_ref, beta_ref, out_ref):
    e = e2_ref[...] + pe_ref[...]
    mean = jnp.mean(e, axis=1, keepdims=True)
    cent = e - mean
    var = jnp.mean(cent * cent, axis=1, keepdims=True)
    rstd = lax.rsqrt(var + EPS)
    out_ref[...] = cent * rstd * gamma_ref[...] + beta_ref[...]


@jax.jit
def _run(pidx, sidx, ln_gamma, ln_beta):
    pe = _pe_table()
    mesh = plsc.VectorSubcoreMesh(core_axis_name="c", subcore_axis_name="s")
    gather_sum = pl.kernel(
        _sc_body,
        out_type=jax.ShapeDtypeStruct((NW * MAX_LEN, DIM), jnp.float32),
        mesh=mesh,
        scratch_types=[
            pltpu.VMEM((NCHUNK, CHUNK), jnp.int32),
            pltpu.VMEM((NCHUNK, CHUNK), jnp.int32),
            pltpu.VMEM((2, CHUNK, DIM), jnp.float32),
            pltpu.VMEM((2, CHUNK, DIM), jnp.float32),
            pltpu.SemaphoreType.DMA((2,)),
            pltpu.SemaphoreType.DMA((2,)),
        ],
    )
    e2 = gather_sum(pe, pidx, sidx)

    ln = pl.pallas_call(
        _tc_ln_body,
        grid=(NW * MAX_LEN // ROWBLK,),
        in_specs=[
            pl.BlockSpec((ROWBLK, DIM), lambda i: (i, 0)),
            pl.BlockSpec((ROWBLK, DIM),
                         lambda i: (i % (MAX_LEN // ROWBLK), 0)),
            pl.BlockSpec((1, DIM), lambda i: (0, 0)),
            pl.BlockSpec((1, DIM), lambda i: (0, 0)),
        ],
        out_specs=pl.BlockSpec((ROWBLK, DIM), lambda i: (i, 0)),
        out_shape=jax.ShapeDtypeStruct((NW * MAX_LEN, DIM), jnp.float32),
    )
    return ln(e2, pe, ln_gamma.reshape(1, DIM), ln_beta.reshape(1, DIM))


def kernel(top_vecs, sent_struct_vec, ln_gamma, ln_beta):
    B, n, _ = top_vecs.shape
    idx = sent_struct_vec.astype(jnp.int32)
    pidx = idx[:, :, 0].reshape(B, NCHUNK, CHUNK)
    sidx = idx[:, :, 1].reshape(B, NCHUNK, CHUNK)
    out = _run(pidx, sidx, ln_gamma.astype(jnp.float32),
               ln_beta.astype(jnp.float32))
    return out.reshape(B, n, DIM)


# speedup vs baseline: 3.1707x; 1.1459x over previous
import functools
import math

import jax
import jax.numpy as jnp
import numpy as np
from jax import lax
from jax.experimental import pallas as pl
from jax.experimental.pallas import tpu as pltpu
from jax.experimental.pallas import tpu_sc as plsc

MAX_LEN = 512
DIM = 1024
EPS = 1e-5

L = 16
NW = 32
CHUNK = 16
NCHUNK = MAX_LEN // CHUNK
ROWBLK = 512


def _pe_table() -> jnp.ndarray:
    position = np.arange(0, MAX_LEN, dtype=np.float32)[:, None]
    div_term = np.exp(
        np.arange(0, DIM, 2, dtype=np.float32) * -(math.log(10000.0) / DIM))
    pe = np.zeros((MAX_LEN, DIM), dtype=np.float32)
    pe[:, 0::2] = np.sin(position * div_term)
    pe[:, 1::2] = np.cos(position * div_term)
    return jnp.asarray(pe)


def _sc_body(pe_hbm, pidx_hbm, sidx_hbm, out_hbm,
             idxp_v, idxs_v, bufp_v, bufs_v, semg, semo):
    wid = lax.axis_index("s") * 2 + lax.axis_index("c")

    pltpu.sync_copy(pidx_hbm.at[wid], idxp_v)
    pltpu.sync_copy(sidx_hbm.at[wid], idxs_v)

    def start_gathers(c, h):
        pltpu.make_async_copy(
            pe_hbm.at[idxp_v.at[c]], bufp_v.at[h], semg.at[h]).start()
        pltpu.make_async_copy(
            pe_hbm.at[idxs_v.at[c]], bufs_v.at[h], semg.at[h]).start()

    def wait_gathers(c, h):
        pltpu.make_async_copy(
            pe_hbm.at[idxp_v.at[c]], bufp_v.at[h], semg.at[h]).wait()
        pltpu.make_async_copy(
            pe_hbm.at[idxs_v.at[c]], bufs_v.at[h], semg.at[h]).wait()

    def out_copy(c, h):
        return pltpu.make_async_copy(
            bufp_v.at[h],
            out_hbm.at[pl.ds(wid * MAX_LEN + c * CHUNK, CHUNK)],
            semo.at[h])

    def compute_chunk(h):
        def row_step(r, _):
            for k in range(DIM // L):
                bufp_v[h, r, pl.ds(k * L, L)] = (
                    bufp_v[h, r, pl.ds(k * L, L)]
                    + bufs_v[h, r, pl.ds(k * L, L)])
            return 0

        lax.fori_loop(0, CHUNK, row_step, 0)

    start_gathers(0, 0)

    def step(i, h):
        c = 2 * i + h

        @pl.when(c >= 1)
        def _():
            out_copy(c - 1, 1 - h).wait()

        @pl.when(c + 1 < NCHUNK)
        def _():
            start_gathers(c + 1, 1 - h)

        wait_gathers(c, h)
        compute_chunk(h)
        out_copy(c, h).start()

    def body(i, _):
        step(i, 0)
        step(i, 1)
        return 0

    lax.fori_loop(0, NCHUNK // 2, body, 0)
    out_copy(NCHUNK - 1, 1).wait()


def _tc_ln_body(e2_ref, pe_ref, gamma_ref, beta_ref, out_ref):
    e = e2_ref[...] + pe_ref[...]
    mean = jnp.mean(e, axis=1, keepdims=True)
    cent = e - mean
    var = jnp.mean(cent * cent, axis=1, keepdims=True)
    rstd = lax.rsqrt(var + EPS)
    out_ref[...] = cent * rstd * gamma_ref[...] + beta_ref[...]


@jax.jit
def _run(pidx, sidx, ln_gamma, ln_beta):
    pe = _pe_table()
    mesh = plsc.VectorSubcoreMesh(core_axis_name="c", subcore_axis_name="s")
    gather_sum = pl.kernel(
        _sc_body,
        out_type=jax.ShapeDtypeStruct((NW * MAX_LEN, DIM), jnp.float32),
        mesh=mesh,
        scratch_types=[
            pltpu.VMEM((NCHUNK, CHUNK), jnp.int32),
            pltpu.VMEM((NCHUNK, CHUNK), jnp.int32),
            pltpu.VMEM((2, CHUNK, DIM), jnp.float32),
            pltpu.VMEM((2, CHUNK, DIM), jnp.float32),
            pltpu.SemaphoreType.DMA((2,)),
            pltpu.SemaphoreType.DMA((2,)),
        ],
    )
    e2 = gather_sum(pe, pidx, sidx)

    ln = pl.pallas_call(
        _tc_ln_body,
        grid=(NW * MAX_LEN // ROWBLK,),
        in_specs=[
            pl.BlockSpec((ROWBLK, DIM), lambda i: (i, 0)),
            pl.BlockSpec((MAX_LEN, DIM), lambda i: (0, 0)),
            pl.BlockSpec((1, DIM), lambda i: (0, 0)),
            pl.BlockSpec((1, DIM), lambda i: (0, 0)),
        ],
        out_specs=pl.BlockSpec((ROWBLK, DIM), lambda i: (i, 0)),
        out_shape=jax.ShapeDtypeStruct((NW * MAX_LEN, DIM), jnp.float32),
    )
    return ln(e2, pe, ln_gamma.reshape(1, DIM), ln_beta.reshape(1, DIM))


def kernel(top_vecs, sent_struct_vec, ln_gamma, ln_beta):
    B, n, _ = top_vecs.shape
    idx = sent_struct_vec.astype(jnp.int32)
    pidx = idx[:, :, 0].reshape(B, NCHUNK, CHUNK)
    sidx = idx[:, :, 1].reshape(B, NCHUNK, CHUNK)
    out = _run(pidx, sidx, ln_gamma.astype(jnp.float32),
               ln_beta.astype(jnp.float32))
    return out.reshape(B, n, DIM)


# speedup vs baseline: 4.0295x; 1.2709x over previous
import functools
import math

import jax
import jax.numpy as jnp
import numpy as np
from jax import lax
from jax.experimental import pallas as pl
from jax.experimental.pallas import tpu as pltpu
from jax.experimental.pallas import tpu_sc as plsc

MAX_LEN = 512
DIM = 1024
EPS = 1e-5

L = 16
NW = 32
CHUNK = 16
NCHUNK = MAX_LEN // CHUNK
ROWBLK = 512


def _pe_table() -> jnp.ndarray:
    position = np.arange(0, MAX_LEN, dtype=np.float32)[:, None]
    div_term = np.exp(
        np.arange(0, DIM, 2, dtype=np.float32) * -(math.log(10000.0) / DIM))
    pe = np.zeros((MAX_LEN, DIM), dtype=np.float32)
    pe[:, 0::2] = np.sin(position * div_term)
    pe[:, 1::2] = np.cos(position * div_term)
    return jnp.asarray(pe)


def _sc_body(pe_hbm, pidx_hbm, sidx_hbm, out_hbm,
             idxp_v, idxs_v, bufp_v, bufs_v, semg, semo):
    wid = lax.axis_index("s") * 2 + lax.axis_index("c")

    pltpu.sync_copy(pidx_hbm.at[wid], idxp_v)
    pltpu.sync_copy(sidx_hbm.at[wid], idxs_v)

    def start_gathers(c, h):
        pltpu.make_async_copy(
            pe_hbm.at[idxp_v.at[c]], bufp_v.at[h], semg.at[h]).start()
        pltpu.make_async_copy(
            pe_hbm.at[idxs_v.at[c]], bufs_v.at[h], semg.at[h]).start()

    def wait_gathers(c, h):
        pltpu.make_async_copy(
            pe_hbm.at[idxp_v.at[c]], bufp_v.at[h], semg.at[h]).wait()
        pltpu.make_async_copy(
            pe_hbm.at[idxs_v.at[c]], bufs_v.at[h], semg.at[h]).wait()

    def out_copy(c, h):
        return pltpu.make_async_copy(
            bufp_v.at[h],
            out_hbm.at[pl.ds(wid * MAX_LEN + c * CHUNK, CHUNK)],
            semo.at[h])

    def compute_chunk(h):
        def row_step(r, _):
            mask = jnp.int32(-65536)
            half = jnp.int32(0x8000)
            for k in range(DIM // (2 * L)):
                wa = bufp_v[h, r, pl.ds(k * L, L)]
                wb = bufs_v[h, r, pl.ds(k * L, L)]
                lo = (lax.bitcast_convert_type(
                          lax.shift_left(wa, 16), jnp.float32)
                      + lax.bitcast_convert_type(
                          lax.shift_left(wb, 16), jnp.float32))
                hi = (lax.bitcast_convert_type(
                          lax.bitwise_and(wa, mask), jnp.float32)
                      + lax.bitcast_convert_type(
                          lax.bitwise_and(wb, mask), jnp.float32))
                lob = lax.bitcast_convert_type(lo, jnp.int32)
                hib = lax.bitcast_convert_type(hi, jnp.int32)
                lw = lax.shift_right_logical(lob + half, 16)
                hw = lax.bitwise_and(hib + half, mask)
                bufp_v[h, r, pl.ds(k * L, L)] = lax.bitwise_or(lw, hw)
            return 0

        lax.fori_loop(0, CHUNK, row_step, 0)

    start_gathers(0, 0)

    def step(i, h):
        c = 2 * i + h

        @pl.when(c >= 1)
        def _():
            out_copy(c - 1, 1 - h).wait()

        @pl.when(c + 1 < NCHUNK)
        def _():
            start_gathers(c + 1, 1 - h)

        wait_gathers(c, h)
        compute_chunk(h)
        out_copy(c, h).start()

    def body(i, _):
        step(i, 0)
        step(i, 1)
        return 0

    lax.fori_loop(0, NCHUNK // 2, body, 0)
    out_copy(NCHUNK - 1, 1).wait()


def _tc_ln_body(e2_ref, pe_ref, gamma_ref, beta_ref, out_ref):
    w = e2_ref[...]
    lo = lax.bitcast_convert_type(lax.shift_left(w, 16), jnp.float32)
    hi = lax.bitcast_convert_type(
        lax.bitwise_and(w, jnp.int32(-65536)), jnp.float32)
    e = jnp.concatenate([lo, hi], axis=1) + pe_ref[...]
    mean = jnp.mean(e, axis=1, keepdims=True)
    cent = e - mean
    var = jnp.mean(cent * cent, axis=1, keepdims=True)
    rstd = lax.rsqrt(var + EPS)
    out_ref[...] = cent * rstd * gamma_ref[...] + beta_ref[...]


@jax.jit
def _run(pidx, sidx, ln_gamma, ln_beta):
    pe = _pe_table()
    pe_bf = pe.astype(jnp.bfloat16)
    pe_w = lax.bitcast_convert_type(
        jnp.stack([pe_bf[:, :DIM // 2], pe_bf[:, DIM // 2:]], axis=-1),
        jnp.int32)
    mesh = plsc.VectorSubcoreMesh(core_axis_name="c", subcore_axis_name="s")
    gather_sum = pl.kernel(
        _sc_body,
        out_type=jax.ShapeDtypeStruct((NW * MAX_LEN, DIM // 2), jnp.int32),
        mesh=mesh,
        scratch_types=[
            pltpu.VMEM((NCHUNK, CHUNK), jnp.int32),
            pltpu.VMEM((NCHUNK, CHUNK), jnp.int32),
            pltpu.VMEM((2, CHUNK, DIM // 2), jnp.int32),
            pltpu.VMEM((2, CHUNK, DIM // 2), jnp.int32),
            pltpu.SemaphoreType.DMA((2,)),
            pltpu.SemaphoreType.DMA((2,)),
        ],
    )
    e2 = gather_sum(pe_w, pidx, sidx)

    ln = pl.pallas_call(
        _tc_ln_body,
        grid=(NW * MAX_LEN // ROWBLK,),
        in_specs=[
            pl.BlockSpec((ROWBLK, DIM // 2), lambda i: (i, 0)),
            pl.BlockSpec((MAX_LEN, DIM), lambda i: (0, 0)),
            pl.BlockSpec((1, DIM), lambda i: (0, 0)),
            pl.BlockSpec((1, DIM), lambda i: (0, 0)),
        ],
        out_specs=pl.BlockSpec((ROWBLK, DIM), lambda i: (i, 0)),
        out_shape=jax.ShapeDtypeStruct((NW * MAX_LEN, DIM), jnp.float32),
    )
    return ln(e2, pe, ln_gamma.reshape(1, DIM), ln_beta.reshape(1, DIM))


def kernel(top_vecs, sent_struct_vec, ln_gamma, ln_beta):
    B, n, _ = top_vecs.shape
    idx = sent_struct_vec.astype(jnp.int32)
    pidx = idx[:, :, 0].reshape(B, NCHUNK, CHUNK)
    sidx = idx[:, :, 1].reshape(B, NCHUNK, CHUNK)
    out = _run(pidx, sidx, ln_gamma.astype(jnp.float32),
               ln_beta.astype(jnp.float32))
    return out.reshape(B, n, DIM)


# speedup vs baseline: 4.3249x; 1.0733x over previous
import functools
import math

import jax
import jax.numpy as jnp
import numpy as np
from jax import lax
from jax.experimental import pallas as pl
from jax.experimental.pallas import tpu as pltpu
from jax.experimental.pallas import tpu_sc as plsc

MAX_LEN = 512
DIM = 1024
EPS = 1e-5

NW = 32
CHUNK = 32
NCHUNK = MAX_LEN // CHUNK
ROWBLK = 512


def _pe_table() -> jnp.ndarray:
    position = np.arange(0, MAX_LEN, dtype=np.float32)[:, None]
    div_term = np.exp(
        np.arange(0, DIM, 2, dtype=np.float32) * -(math.log(10000.0) / DIM))
    pe = np.zeros((MAX_LEN, DIM), dtype=np.float32)
    pe[:, 0::2] = np.sin(position * div_term)
    pe[:, 1::2] = np.cos(position * div_term)
    return jnp.asarray(pe)


def _sc_body(pe_hbm, pidx_hbm, sidx_hbm, out_hbm,
             idxp_v, idxs_v, bufp_v, bufs_v, semg, semo):
    wid = lax.axis_index("s") * 2 + lax.axis_index("c")

    pltpu.sync_copy(pidx_hbm.at[wid], idxp_v)
    pltpu.sync_copy(sidx_hbm.at[wid], idxs_v)

    def start_gathers(c, h):
        pltpu.make_async_copy(
            pe_hbm.at[idxp_v.at[c]], bufp_v.at[h], semg.at[h]).start()
        pltpu.make_async_copy(
            pe_hbm.at[idxs_v.at[c]], bufs_v.at[h], semg.at[h]).start()

    def wait_gathers(c, h):
        pltpu.make_async_copy(
            pe_hbm.at[idxp_v.at[c]], bufp_v.at[h], semg.at[h]).wait()
        pltpu.make_async_copy(
            pe_hbm.at[idxs_v.at[c]], bufs_v.at[h], semg.at[h]).wait()

    def rows(c):
        return pl.ds(wid * MAX_LEN + c * CHUNK, CHUNK)

    def start_out(c, h):
        pltpu.make_async_copy(
            bufp_v.at[h], out_hbm.at[0, rows(c)], semo.at[h]).start()
        pltpu.make_async_copy(
            bufs_v.at[h], out_hbm.at[1, rows(c)], semo.at[h]).start()

    def wait_out(c, h):
        pltpu.make_async_copy(
            bufp_v.at[h], out_hbm.at[0, rows(c)], semo.at[h]).wait()
        pltpu.make_async_copy(
            bufs_v.at[h], out_hbm.at[1, rows(c)], semo.at[h]).wait()

    start_gathers(0, 0)

    def step(i, h):
        c = 2 * i + h
        wait_gathers(c, h)
        start_out(c, h)

        @pl.when(c >= 1)
        def _():
            wait_out(c - 1, 1 - h)

        @pl.when(c + 1 < NCHUNK)
        def _():
            start_gathers(c + 1, 1 - h)

    def body(i, _):
        step(i, 0)
        step(i, 1)
        return 0

    lax.fori_loop(0, NCHUNK // 2, body, 0)
    wait_out(NCHUNK - 1, 1)


def _tc_ln_body(wp_ref, ws_ref, pe_ref, gamma_ref, beta_ref, out_ref):
    wp = wp_ref[0]
    ws = ws_ref[0]
    mask = jnp.int32(-65536)
    lo = (lax.bitcast_convert_type(lax.shift_left(wp, 16), jnp.float32)
          + lax.bitcast_convert_type(lax.shift_left(ws, 16), jnp.float32))
    hi = (lax.bitcast_convert_type(lax.bitwise_and(wp, mask), jnp.float32)
          + lax.bitcast_convert_type(lax.bitwise_and(ws, mask), jnp.float32))
    e = jnp.concatenate([lo, hi], axis=1) + pe_ref[...]
    mean = jnp.mean(e, axis=1, keepdims=True)
    cent = e - mean
    var = jnp.mean(cent * cent, axis=1, keepdims=True)
    rstd = lax.rsqrt(var + EPS)
    out_ref[...] = cent * rstd * gamma_ref[...] + beta_ref[...]


@jax.jit
def _run(pidx, sidx, ln_gamma, ln_beta):
    pe = _pe_table()
    pe_bf = pe.astype(jnp.bfloat16)
    pe_w = lax.bitcast_convert_type(
        jnp.stack([pe_bf[:, :DIM // 2], pe_bf[:, DIM // 2:]], axis=-1),
        jnp.int32)
    mesh = plsc.VectorSubcoreMesh(core_axis_name="c", subcore_axis_name="s")
    gather2 = pl.kernel(
        _sc_body,
        out_type=jax.ShapeDtypeStruct((2, NW * MAX_LEN, DIM // 2), jnp.int32),
        mesh=mesh,
        scratch_types=[
            pltpu.VMEM((NCHUNK, CHUNK), jnp.int32),
            pltpu.VMEM((NCHUNK, CHUNK), jnp.int32),
            pltpu.VMEM((2, CHUNK, DIM // 2), jnp.int32),
            pltpu.VMEM((2, CHUNK, DIM // 2), jnp.int32),
            pltpu.SemaphoreType.DMA((2,)),
            pltpu.SemaphoreType.DMA((2,)),
        ],
    )
    e2 = gather2(pe_w, pidx, sidx)

    ln = pl.pallas_call(
        _tc_ln_body,
        grid=(NW * MAX_LEN // ROWBLK,),
        in_specs=[
            pl.BlockSpec((1, ROWBLK, DIM // 2), lambda i: (0, i, 0)),
            pl.BlockSpec((1, ROWBLK, DIM // 2), lambda i: (1, i, 0)),
            pl.BlockSpec((MAX_LEN, DIM), lambda i: (0, 0)),
            pl.BlockSpec((1, DIM), lambda i: (0, 0)),
            pl.BlockSpec((1, DIM), lambda i: (0, 0)),
        ],
        out_specs=pl.BlockSpec((ROWBLK, DIM), lambda i: (i, 0)),
        out_shape=jax.ShapeDtypeStruct((NW * MAX_LEN, DIM), jnp.float32),
    )
    return ln(e2, e2, pe, ln_gamma.reshape(1, DIM), ln_beta.reshape(1, DIM))


def kernel(top_vecs, sent_struct_vec, ln_gamma, ln_beta):
    B, n, _ = top_vecs.shape
    idx = sent_struct_vec.astype(jnp.int32)
    pidx = idx[:, :, 0].reshape(B, NCHUNK, CHUNK)
    sidx = idx[:, :, 1].reshape(B, NCHUNK, CHUNK)
    out = _run(pidx, sidx, ln_gamma.astype(jnp.float32),
               ln_beta.astype(jnp.float32))
    return out.reshape(B, n, DIM)


# speedup vs baseline: 4.4497x; 1.0289x over previous
import functools
import math

import jax
import jax.numpy as jnp
import numpy as np
from jax import lax
from jax.experimental import pallas as pl
from jax.experimental.pallas import tpu as pltpu
from jax.experimental.pallas import tpu_sc as plsc

MAX_LEN = 512
DIM = 1024
EPS = 1e-5

NW = 32
CHUNK = 32
ROWS = NW * MAX_LEN
HALF_ROWS = ROWS // 2
RPW = HALF_ROWS // NW
NCHUNK = RPW // CHUNK
ROWBLK = 512
NBLK_H = HALF_ROWS // ROWBLK


def _pe_table() -> jnp.ndarray:
    position = np.arange(0, MAX_LEN, dtype=np.float32)[:, None]
    div_term = np.exp(
        np.arange(0, DIM, 2, dtype=np.float32) * -(math.log(10000.0) / DIM))
    pe = np.zeros((MAX_LEN, DIM), dtype=np.float32)
    pe[:, 0::2] = np.sin(position * div_term)
    pe[:, 1::2] = np.cos(position * div_term)
    return jnp.asarray(pe)


def _sc_body(pe_hbm, pidx_hbm, sidx_hbm, out_hbm,
             idxp_v, idxs_v, bufp_v, bufs_v, semg, semo):
    wid = lax.axis_index("s") * 2 + lax.axis_index("c")

    pltpu.sync_copy(pidx_hbm.at[wid], idxp_v)
    pltpu.sync_copy(sidx_hbm.at[wid], idxs_v)

    def start_gathers(c, h):
        pltpu.make_async_copy(
            pe_hbm.at[idxp_v.at[c]], bufp_v.at[h], semg.at[h]).start()
        pltpu.make_async_copy(
            pe_hbm.at[idxs_v.at[c]], bufs_v.at[h], semg.at[h]).start()

    def wait_gathers(c, h):
        pltpu.make_async_copy(
            pe_hbm.at[idxp_v.at[c]], bufp_v.at[h], semg.at[h]).wait()
        pltpu.make_async_copy(
            pe_hbm.at[idxs_v.at[c]], bufs_v.at[h], semg.at[h]).wait()

    def rows(c):
        return pl.ds(wid * RPW + c * CHUNK, CHUNK)

    def start_out(c, h):
        pltpu.make_async_copy(
            bufp_v.at[h], out_hbm.at[0, rows(c)], semo.at[h]).start()
        pltpu.make_async_copy(
            bufs_v.at[h], out_hbm.at[1, rows(c)], semo.at[h]).start()

    def wait_out(c, h):
        pltpu.make_async_copy(
            bufp_v.at[h], out_hbm.at[0, rows(c)], semo.at[h]).wait()
        pltpu.make_async_copy(
            bufs_v.at[h], out_hbm.at[1, rows(c)], semo.at[h]).wait()

    start_gathers(0, 0)

    def step(i, h):
        c = 2 * i + h
        wait_gathers(c, h)
        start_out(c, h)

        @pl.when(c >= 1)
        def _():
            wait_out(c - 1, 1 - h)

        @pl.when(c + 1 < NCHUNK)
        def _():
            start_gathers(c + 1, 1 - h)

    def body(i, _):
        step(i, 0)
        step(i, 1)
        return 0

    lax.fori_loop(0, NCHUNK // 2, body, 0)
    wait_out(NCHUNK - 1, 1)


def _ln_block(wp, ws, pe, gamma, beta):
    mask = jnp.int32(-65536)
    lo = (lax.bitcast_convert_type(lax.shift_left(wp, 16), jnp.float32)
          + lax.bitcast_convert_type(lax.shift_left(ws, 16), jnp.float32))
    hi = (lax.bitcast_convert_type(lax.bitwise_and(wp, mask), jnp.float32)
          + lax.bitcast_convert_type(lax.bitwise_and(ws, mask), jnp.float32))
    e = jnp.concatenate([lo, hi], axis=1) + pe
    mean = jnp.mean(e, axis=1, keepdims=True)
    cent = e - mean
    var = jnp.mean(cent * cent, axis=1, keepdims=True)
    rstd = lax.rsqrt(var + EPS)
    return cent * rstd * gamma + beta


def _tc_ln_body(wp_ref, ws_ref, pe_ref, gamma_ref, beta_ref, out_ref):
    out_ref[...] = _ln_block(wp_ref[0], ws_ref[0], pe_ref[...],
                             gamma_ref[...], beta_ref[...])


def _tc_ln_alias_body(acc_ref, wp_ref, ws_ref, pe_ref, gamma_ref, beta_ref,
                      out_ref):
    del acc_ref
    out_ref[...] = _ln_block(wp_ref[0], ws_ref[0], pe_ref[...],
                             gamma_ref[...], beta_ref[...])


@jax.jit
def _run(pidx, sidx, ln_gamma, ln_beta):
    pe = _pe_table()
    pe_bf = pe.astype(jnp.bfloat16)
    pe_w = lax.bitcast_convert_type(
        jnp.stack([pe_bf[:, :DIM // 2], pe_bf[:, DIM // 2:]], axis=-1),
        jnp.int32)
    mesh = plsc.VectorSubcoreMesh(core_axis_name="c", subcore_axis_name="s")
    gather2 = pl.kernel(
        _sc_body,
        out_type=jax.ShapeDtypeStruct((2, HALF_ROWS, DIM // 2), jnp.int32),
        mesh=mesh,
        scratch_types=[
            pltpu.VMEM((NCHUNK, CHUNK), jnp.int32),
            pltpu.VMEM((NCHUNK, CHUNK), jnp.int32),
            pltpu.VMEM((2, CHUNK, DIM // 2), jnp.int32),
            pltpu.VMEM((2, CHUNK, DIM // 2), jnp.int32),
            pltpu.SemaphoreType.DMA((2,)),
            pltpu.SemaphoreType.DMA((2,)),
        ],
    )

    e2_specs = [
        pl.BlockSpec((1, ROWBLK, DIM // 2), lambda i: (0, i, 0)),
        pl.BlockSpec((1, ROWBLK, DIM // 2), lambda i: (1, i, 0)),
    ]
    fixed_specs = [
        pl.BlockSpec((MAX_LEN, DIM), lambda i: (0, 0)),
        pl.BlockSpec((1, DIM), lambda i: (0, 0)),
        pl.BlockSpec((1, DIM), lambda i: (0, 0)),
    ]
    out_shape = jax.ShapeDtypeStruct((ROWS, DIM), jnp.float32)
    ln0 = pl.pallas_call(
        _tc_ln_body,
        grid=(NBLK_H,),
        in_specs=e2_specs + fixed_specs,
        out_specs=pl.BlockSpec((ROWBLK, DIM), lambda i: (i, 0)),
        out_shape=out_shape,
    )
    ln1 = pl.pallas_call(
        _tc_ln_alias_body,
        grid=(NBLK_H,),
        in_specs=[pl.BlockSpec(memory_space=pl.ANY)]
        + e2_specs + fixed_specs,
        out_specs=pl.BlockSpec((ROWBLK, DIM), lambda i: (i + NBLK_H, 0)),
        out_shape=out_shape,
        input_output_aliases={0: 0},
    )

    gamma2 = ln_gamma.reshape(1, DIM)
    beta2 = ln_beta.reshape(1, DIM)
    e2_0 = gather2(pe_w, pidx[0], sidx[0])
    e2_1 = gather2(pe_w, pidx[1], sidx[1])
    acc = ln0(e2_0, e2_0, pe, gamma2, beta2)
    return ln1(acc, e2_1, e2_1, pe, gamma2, beta2)


def kernel(top_vecs, sent_struct_vec, ln_gamma, ln_beta):
    B, n, _ = top_vecs.shape
    idx = sent_struct_vec.astype(jnp.int32)
    pidx = idx[:, :, 0].reshape(2, NW, NCHUNK, CHUNK)
    sidx = idx[:, :, 1].reshape(2, NW, NCHUNK, CHUNK)
    out = _run(pidx, sidx, ln_gamma.astype(jnp.float32),
               ln_beta.astype(jnp.float32))
    return out.reshape(B, n, DIM)


# speedup vs baseline: 4.5569x; 1.0241x over previous
import functools
import math

import jax
import jax.numpy as jnp
import numpy as np
from jax import lax
from jax.experimental import pallas as pl
from jax.experimental.pallas import tpu as pltpu
from jax.experimental.pallas import tpu_sc as plsc

MAX_LEN = 512
DIM = 1024
EPS = 1e-5

L = 16
NW = 32
CHUNK = 32
ROWS = NW * MAX_LEN
HALF_ROWS = ROWS // 2
RPW = HALF_ROWS // NW
NCHUNK = RPW // CHUNK
ROWBLK = 512
NBLK_H = HALF_ROWS // ROWBLK


def _pe_table() -> jnp.ndarray:
    position = np.arange(0, MAX_LEN, dtype=np.float32)[:, None]
    div_term = np.exp(
        np.arange(0, DIM, 2, dtype=np.float32) * -(math.log(10000.0) / DIM))
    pe = np.zeros((MAX_LEN, DIM), dtype=np.float32)
    pe[:, 0::2] = np.sin(position * div_term)
    pe[:, 1::2] = np.cos(position * div_term)
    return jnp.asarray(pe)


def _sc_body(pe_hbm, pidx_hbm, sidx_hbm, out_hbm,
             idxp_v, idxs_v, bufp_v, bufs_v, semg, semo):
    wid = lax.axis_index("s") * 2 + lax.axis_index("c")

    pltpu.sync_copy(pidx_hbm.at[wid], idxp_v)
    pltpu.sync_copy(sidx_hbm.at[wid], idxs_v)

    def start_gathers(c, h):
        pltpu.make_async_copy(
            pe_hbm.at[idxp_v.at[c]], bufp_v.at[h], semg.at[h]).start()
        pltpu.make_async_copy(
            pe_hbm.at[idxs_v.at[c]], bufs_v.at[h], semg.at[h]).start()

    def wait_gathers(c, h):
        pltpu.make_async_copy(
            pe_hbm.at[idxp_v.at[c]], bufp_v.at[h], semg.at[h]).wait()
        pltpu.make_async_copy(
            pe_hbm.at[idxs_v.at[c]], bufs_v.at[h], semg.at[h]).wait()

    def rows(c):
        return pl.ds(wid * RPW + c * CHUNK, CHUNK)

    def out_copy(c, h):
        return pltpu.make_async_copy(
            bufp_v.at[h], out_hbm.at[rows(c)], semo.at[h])

    def compute_chunk(h):
        def row_step(r, _):
            mask = jnp.int32(-65536)
            for k in range(DIM // (2 * L)):
                wa = bufp_v[h, r, pl.ds(k * L, L)]
                wb = bufs_v[h, r, pl.ds(k * L, L)]
                lo = (lax.bitcast_convert_type(
                          lax.shift_left(wa, 16), jnp.float32)
                      + lax.bitcast_convert_type(
                          lax.shift_left(wb, 16), jnp.float32))
                hi = (lax.bitcast_convert_type(
                          lax.bitwise_and(wa, mask), jnp.float32)
                      + lax.bitcast_convert_type(
                          lax.bitwise_and(wb, mask), jnp.float32))
                lw = lax.shift_right_logical(
                    lax.bitcast_convert_type(lo, jnp.int32), 16)
                hw = lax.bitwise_and(
                    lax.bitcast_convert_type(hi, jnp.int32), mask)
                bufp_v[h, r, pl.ds(k * L, L)] = lax.bitwise_or(lw, hw)
            return 0

        lax.fori_loop(0, CHUNK, row_step, 0)

    start_gathers(0, 0)

    def step(i, h):
        c = 2 * i + h

        @pl.when(c >= 1)
        def _():
            out_copy(c - 1, 1 - h).wait()

        @pl.when(c + 1 < NCHUNK)
        def _():
            start_gathers(c + 1, 1 - h)

        wait_gathers(c, h)
        compute_chunk(h)
        out_copy(c, h).start()

    def body(i, _):
        step(i, 0)
        step(i, 1)
        return 0

    lax.fori_loop(0, NCHUNK // 2, body, 0)
    out_copy(NCHUNK - 1, 1).wait()


def _ln_block(w, pe, gamma, beta):
    mask = jnp.int32(-65536)
    lo = lax.bitcast_convert_type(lax.shift_left(w, 16), jnp.float32)
    hi = lax.bitcast_convert_type(lax.bitwise_and(w, mask), jnp.float32)
    e = jnp.concatenate([lo, hi], axis=1) + pe
    mean = jnp.mean(e, axis=1, keepdims=True)
    cent = e - mean
    var = jnp.mean(cent * cent, axis=1, keepdims=True)
    rstd = lax.rsqrt(var + EPS)
    return cent * rstd * gamma + beta


def _tc_ln_body(w_ref, pe_ref, gamma_ref, beta_ref, out_ref):
    out_ref[...] = _ln_block(w_ref[...], pe_ref[...],
                             gamma_ref[...], beta_ref[...])


def _tc_ln_alias_body(acc_ref, w_ref, pe_ref, gamma_ref, beta_ref,
                      out_ref):
    del acc_ref
    out_ref[...] = _ln_block(w_ref[...], pe_ref[...],
                             gamma_ref[...], beta_ref[...])


@jax.jit
def _run(pidx, sidx, ln_gamma, ln_beta):
    pe = _pe_table()
    pe_bf = pe.astype(jnp.bfloat16)
    pe_w = lax.bitcast_convert_type(
        jnp.stack([pe_bf[:, :DIM // 2], pe_bf[:, DIM // 2:]], axis=-1),
        jnp.int32)
    mesh = plsc.VectorSubcoreMesh(core_axis_name="c", subcore_axis_name="s")
    gather2 = pl.kernel(
        _sc_body,
        out_type=jax.ShapeDtypeStruct((HALF_ROWS, DIM // 2), jnp.int32),
        mesh=mesh,
        scratch_types=[
            pltpu.VMEM((NCHUNK, CHUNK), jnp.int32),
            pltpu.VMEM((NCHUNK, CHUNK), jnp.int32),
            pltpu.VMEM((2, CHUNK, DIM // 2), jnp.int32),
            pltpu.VMEM((2, CHUNK, DIM // 2), jnp.int32),
            pltpu.SemaphoreType.DMA((2,)),
            pltpu.SemaphoreType.DMA((2,)),
        ],
    )

    e2_specs = [
        pl.BlockSpec((ROWBLK, DIM // 2), lambda i: (i, 0)),
    ]
    fixed_specs = [
        pl.BlockSpec((MAX_LEN, DIM), lambda i: (0, 0)),
        pl.BlockSpec((1, DIM), lambda i: (0, 0)),
        pl.BlockSpec((1, DIM), lambda i: (0, 0)),
    ]
    out_shape = jax.ShapeDtypeStruct((ROWS, DIM), jnp.float32)
    ln0 = pl.pallas_call(
        _tc_ln_body,
        grid=(NBLK_H,),
        in_specs=e2_specs + fixed_specs,
        out_specs=pl.BlockSpec((ROWBLK, DIM), lambda i: (i, 0)),
        out_shape=out_shape,
    )
    ln1 = pl.pallas_call(
        _tc_ln_alias_body,
        grid=(NBLK_H,),
        in_specs=[pl.BlockSpec(memory_space=pl.ANY)]
        + e2_specs + fixed_specs,
        out_specs=pl.BlockSpec((ROWBLK, DIM), lambda i: (i + NBLK_H, 0)),
        out_shape=out_shape,
        input_output_aliases={0: 0},
    )

    gamma2 = ln_gamma.reshape(1, DIM)
    beta2 = ln_beta.reshape(1, DIM)
    e2_0 = gather2(pe_w, pidx[0], sidx[0])
    e2_1 = gather2(pe_w, pidx[1], sidx[1])
    acc = ln0(e2_0, pe, gamma2, beta2)
    return ln1(acc, e2_1, pe, gamma2, beta2)


def kernel(top_vecs, sent_struct_vec, ln_gamma, ln_beta):
    B, n, _ = top_vecs.shape
    idx = sent_struct_vec.astype(jnp.int32)
    pidx = idx[:, :, 0].reshape(2, NW, NCHUNK, CHUNK)
    sidx = idx[:, :, 1].reshape(2, NW, NCHUNK, CHUNK)
    out = _run(pidx, sidx, ln_gamma.astype(jnp.float32),
               ln_beta.astype(jnp.float32))
    return out.reshape(B, n, DIM)
